# Initial kernel scaffold; baseline (speedup 1.0000x reference)
import jax
import jax.numpy as jnp
from jax import lax
from jax.experimental import pallas as pl
from jax.experimental.pallas import tpu as pltpu
from jax.experimental.pallas import tpu_sc as plsc

EMBED_DIM = 32
BATCH = 16384

_INFO = plsc.get_sparse_core_info()
_NC = _INFO.num_cores
_NS = _INFO.num_subcores
_NW = _NC * _NS
_CHUNK = 128
_BPW = BATCH // _NW
_NCH = _BPW // _CHUNK


def _body(uid_idx, loc_idx, emb_uid, emb_loc, out, idxu_v, idxl_v,
          u_rows, l_rows, sem):
  wid = lax.axis_index("s") * _NC + lax.axis_index("c")
  base = wid * _BPW
  pltpu.sync_copy(uid_idx.at[pl.ds(wid * _NCH, _NCH)], idxu_v)
  pltpu.sync_copy(loc_idx.at[pl.ds(wid * _NCH, _NCH)], idxl_v)
  copies = []
  for j in range(_NCH):
    dst = pl.ds(j * _CHUNK, _CHUNK)
    copies.append(pltpu.async_copy(emb_uid.at[idxu_v.at[j]],
                                   u_rows.at[dst], sem))
    copies.append(pltpu.async_copy(emb_loc.at[idxl_v.at[j]],
                                   l_rows.at[dst], sem))
  for cp in copies:
    cp.wait()
  pltpu.sync_copy(u_rows, out.at[pl.ds(base, _BPW), pl.ds(0, EMBED_DIM)])
  pltpu.sync_copy(l_rows, out.at[pl.ds(base, _BPW), pl.ds(EMBED_DIM, EMBED_DIM)])


@jax.jit
def _lookup(uid_idx, loc_idx, emb_uid, emb_loc):
  mesh = plsc.VectorSubcoreMesh(core_axis_name="c", subcore_axis_name="s")
  return pl.kernel(
      _body,
      out_type=jax.ShapeDtypeStruct((BATCH, 2 * EMBED_DIM), jnp.float32),
      mesh=mesh,
      scratch_types=[
          pltpu.VMEM((_NCH, _CHUNK), jnp.int32),
          pltpu.VMEM((_NCH, _CHUNK), jnp.int32),
          pltpu.VMEM((_BPW, EMBED_DIM), jnp.float32),
          pltpu.VMEM((_BPW, EMBED_DIM), jnp.float32),
          pltpu.SemaphoreType.DMA,
      ],
  )(uid_idx, loc_idx, emb_uid, emb_loc)


def kernel(user_fea, emb_uid, emb_loc):
  uid_idx = user_fea[:, 0].reshape(BATCH // _CHUNK, _CHUNK)
  loc_idx = user_fea[:, 1].reshape(BATCH // _CHUNK, _CHUNK)
  return _lookup(uid_idx, loc_idx, emb_uid, emb_loc)


# speedup vs baseline: 1.2417x; 1.2417x over previous
import jax
import jax.numpy as jnp
from jax import lax
from jax.experimental import pallas as pl
from jax.experimental.pallas import tpu as pltpu
from jax.experimental.pallas import tpu_sc as plsc

EMBED_DIM = 32
BATCH = 16384

_INFO = plsc.get_sparse_core_info()
_NC = _INFO.num_cores
_NS = _INFO.num_subcores
_NW = _NC * _NS
_CHUNK = 128
_BPW = BATCH // _NW
_NCH = _BPW // _CHUNK


def _body(uid_idx, loc_idx, emb_uid, emb_loc, out, idxu_v, idxl_v,
          u_rows, l_rows, sem):
  wid = lax.axis_index("s") * _NC + lax.axis_index("c")
  base = wid * _BPW
  pltpu.sync_copy(uid_idx.at[pl.ds(wid * _NCH, _NCH)], idxu_v)
  pltpu.sync_copy(loc_idx.at[pl.ds(wid * _NCH, _NCH)], idxl_v)
  copies = []
  for j in range(_NCH):
    dst = pl.ds(j * _CHUNK, _CHUNK)
    copies.append(pltpu.async_copy(emb_uid.at[idxu_v.at[j]],
                                   u_rows.at[dst], sem))
    copies.append(pltpu.async_copy(emb_loc.at[idxl_v.at[j]],
                                   l_rows.at[dst], sem))
  for cp in copies:
    cp.wait()
  pltpu.sync_copy(u_rows, out.at[pl.ds(base, _BPW), pl.ds(0, EMBED_DIM)])
  pltpu.sync_copy(l_rows, out.at[pl.ds(base, _BPW), pl.ds(EMBED_DIM, EMBED_DIM)])


@jax.jit
def _lookup(uid_idx, loc_idx, emb_uid, emb_loc):
  mesh = plsc.VectorSubcoreMesh(core_axis_name="c", subcore_axis_name="s")
  return pl.kernel(
      _body,
      out_type=jax.ShapeDtypeStruct((BATCH, 2 * EMBED_DIM), jnp.float32),
      mesh=mesh,
      compiler_params=pltpu.CompilerParams(use_tc_tiling_on_sc=False),
      scratch_types=[
          pltpu.VMEM((_NCH, _CHUNK), jnp.int32),
          pltpu.VMEM((_NCH, _CHUNK), jnp.int32),
          pltpu.VMEM((_BPW, EMBED_DIM), jnp.float32),
          pltpu.VMEM((_BPW, EMBED_DIM), jnp.float32),
          pltpu.SemaphoreType.DMA,
      ],
  )(uid_idx, loc_idx, emb_uid, emb_loc)


def kernel(user_fea, emb_uid, emb_loc):
  uid_idx = user_fea[:, 0].reshape(BATCH // _CHUNK, _CHUNK)
  loc_idx = user_fea[:, 1].reshape(BATCH // _CHUNK, _CHUNK)
  return _lookup(uid_idx, loc_idx, emb_uid, emb_loc)
